# baseline (speedup 1.0000x reference)
import functools

import jax
import jax.numpy as jnp
from jax import lax
from jax.experimental import pallas as pl
from jax.experimental.pallas import tpu as pltpu
from jax.experimental.pallas import tpu_sc as plsc

DIM = 64
SCALE = 8.0
NC, NS, LANES = 2, 16, 16
NW = NC * NS
NBUF = 4
XBUF = 2
G = 2


def kernel(tokens, table):
    B, L = tokens.shape
    BW = B // NW
    nch = L // G
    nt = nch // NBUF
    CT = DIM // 8

    idx3 = jnp.transpose(
        tokens.astype(jnp.int32).reshape(NW, BW, L), (0, 2, 1)
    )

    mesh = plsc.VectorSubcoreMesh(core_axis_name="c", subcore_axis_name="s")

    @functools.partial(
        pl.kernel,
        mesh=mesh,
        compiler_params=pltpu.CompilerParams(
            use_tc_tiling_on_sc=False, needs_layout_passes=False
        ),
        out_type=jax.ShapeDtypeStruct((L, CT, NW, 8, LANES * 8), jnp.float32),
        scratch_types=[
            pltpu.VMEM((L, BW), jnp.int32),
            pltpu.VMEM((NBUF, G * BW, DIM), jnp.float32),
            pltpu.VMEM((XBUF, G, DIM, BW), jnp.float32),
        ]
        + [pltpu.SemaphoreType.DMA] * (NBUF + XBUF),
    )
    def emb_kernel(idx_hbm, tab_hbm, out_hbm, idx_v, raw_v, xout_v, *sems):
        gsem = sems[:NBUF]
        osem = sems[NBUF:]
        wid = lax.axis_index("s") * NC + lax.axis_index("c")
        pltpu.sync_copy(idx_hbm.at[wid], idx_v)
        lane_iota = lax.iota(jnp.int32, LANES)

        def gather(c, b):
            return [
                pltpu.make_async_copy(
                    tab_hbm.at[idx_v.at[c * G + j]],
                    raw_v.at[b].at[pl.ds(j * BW, BW)],
                    gsem[b],
                )
                for j in range(G)
            ]

        def slab_stores(c, xb):
            return [
                pltpu.make_async_copy(
                    xout_v.at[xb].at[j].at[pl.ds(ct * 8, 8)],
                    out_hbm.at[c * G + j, ct, wid],
                    osem[xb],
                )
                for j in range(G)
                for ct in range(CT)
            ]

        for cp in gather(0, 0):
            cp.start()
        for cp in gather(1, 1):
            cp.start()

        @pl.loop(0, nt)
        def outer(t):
            for b in range(NBUF):
                c = t * NBUF + b
                xb = b % XBUF
                if b < 2:
                    for cp in gather(c + 2, (b + 2) % NBUF):
                        cp.start()
                else:
                    @pl.when(t < nt - 1)
                    def _():
                        for cp in gather(c + 2, (b + 2) % NBUF):
                            cp.start()

                for cp in gather(c, b):
                    cp.wait()

                if b < 2:
                    @pl.when(t > 0)
                    def _():
                        for cp in slab_stores(c - 2, xb):
                            cp.wait()
                else:
                    for cp in slab_stores(c - 2, xb):
                        cp.wait()

                for j in range(G):
                    @plsc.parallel_loop(0, DIM, 1, unroll=4)
                    def transpose_scale(c64):
                        cols = lane_iota * 0 + c64
                        for k in range(BW // LANES):
                            rows = lane_iota + (j * BW + k * LANES)
                            v = plsc.load_gather(raw_v.at[b], [rows, cols])
                            xout_v[xb, j, c64, pl.ds(k * LANES, LANES)] = (
                                v * SCALE
                            )

                for cp in slab_stores(c, xb):
                    cp.start()

        for cc, xb in ((nch - 2, 0), (nch - 1, 1)):
            for cp in slab_stores(cc, xb):
                cp.wait()

    out5 = emb_kernel(idx3, table)
    return jnp.transpose(out5, (2, 4, 0, 1, 3)).reshape(B, L, DIM)


# speedup vs baseline: 1.4368x; 1.4368x over previous
import functools

import jax
import jax.numpy as jnp
from jax import lax
from jax.experimental import pallas as pl
from jax.experimental.pallas import tpu as pltpu
from jax.experimental.pallas import tpu_sc as plsc

DIM = 64
SCALE = 8.0
NC, NS, LANES = 2, 16, 16
NW = NC * NS
NBUF = 4
XBUF = 2
G = 2


def kernel(tokens, table):
    B, L = tokens.shape
    BW = B // NW
    nch = L // G
    nt = nch // NBUF
    CT = DIM // 8

    idx3 = jnp.transpose(
        tokens.astype(jnp.int32).reshape(NW, BW, L), (0, 2, 1)
    )

    mesh = plsc.VectorSubcoreMesh(core_axis_name="c", subcore_axis_name="s")

    @functools.partial(
        pl.kernel,
        mesh=mesh,
        compiler_params=pltpu.CompilerParams(
            use_tc_tiling_on_sc=False, needs_layout_passes=False
        ),
        out_type=jax.ShapeDtypeStruct((L, CT, NW, 8, LANES * 8), jnp.float32),
        scratch_types=[
            pltpu.VMEM((L, BW), jnp.int32),
            pltpu.VMEM((NBUF, G * BW, DIM), jnp.float32),
            pltpu.VMEM((XBUF, G, DIM, BW), jnp.float32),
        ]
        + [pltpu.SemaphoreType.DMA] * (NBUF + XBUF),
    )
    def emb_kernel(idx_hbm, tab_hbm, out_hbm, idx_v, raw_v, xout_v, *sems):
        gsem = sems[:NBUF]
        osem = sems[NBUF:]
        wid = lax.axis_index("s") * NC + lax.axis_index("c")
        pltpu.sync_copy(idx_hbm.at[wid], idx_v)
        lane_iota = lax.iota(jnp.int32, LANES)

        def gather(c, b):
            return [
                pltpu.make_async_copy(
                    tab_hbm.at[idx_v.at[c * G + j]],
                    raw_v.at[b].at[pl.ds(j * BW, BW)],
                    gsem[b],
                )
                for j in range(G)
            ]

        def slab_stores(c, xb):
            return [
                pltpu.make_async_copy(
                    xout_v.at[xb].at[j].at[pl.ds(ct * 8, 8)],
                    out_hbm.at[c * G + j, ct, wid],
                    osem[xb],
                )
                for j in range(G)
                for ct in range(CT)
            ]

        for cp in gather(0, 0):
            cp.start()
        for cp in gather(1, 1):
            cp.start()

        @pl.loop(0, nt)
        def outer(t):
            for b in range(NBUF):
                c = t * NBUF + b
                xb = b % XBUF
                if b < 2:
                    for cp in gather(c + 2, (b + 2) % NBUF):
                        cp.start()
                else:
                    @pl.when(t < nt - 1)
                    def _():
                        for cp in gather(c + 2, (b + 2) % NBUF):
                            cp.start()

                for cp in gather(c, b):
                    cp.wait()

                if b < 2:
                    @pl.when(t > 0)
                    def _():
                        for cp in slab_stores(c - 2, xb):
                            cp.wait()
                else:
                    for cp in slab_stores(c - 2, xb):
                        cp.wait()

                @plsc.parallel_loop(0, LANES * G * 32, 1, unroll=4)
                def transpose_scale(q):
                    s = q & 15
                    blk = q >> 4
                    j = blk >> 5
                    rem = blk & 31
                    c0 = (rem >> 3) * LANES
                    r0 = (rem & 7) * LANES
                    rot = (lane_iota + s) & 15
                    rows = lane_iota + (r0 + j * BW)
                    cols = rot + c0
                    v = plsc.load_gather(raw_v.at[b], [rows, cols])
                    plsc.store_scatter(
                        xout_v.at[xb],
                        [cols * 0 + j, cols, lane_iota + r0],
                        v * SCALE,
                    )

                for cp in slab_stores(c, xb):
                    cp.start()

        for cc, xb in ((nch - 2, 0), (nch - 1, 1)):
            for cp in slab_stores(cc, xb):
                cp.wait()

    out5 = emb_kernel(idx3, table)
    return jnp.transpose(out5, (2, 4, 0, 1, 3)).reshape(B, L, DIM)


# speedup vs baseline: 1.4671x; 1.0211x over previous
import functools

import jax
import jax.numpy as jnp
from jax import lax
from jax.experimental import pallas as pl
from jax.experimental.pallas import tpu as pltpu
from jax.experimental.pallas import tpu_sc as plsc

DIM = 64
SCALE = 8.0
NC, NS, LANES = 2, 16, 16
NW = NC * NS
NBUF = 4
XBUF = 2
G = 2


def kernel(tokens, table):
    B, L = tokens.shape
    BW = B // NW
    nch = L // G
    nt = nch // NBUF
    CT = DIM // 8

    idx3 = jnp.transpose(
        tokens.astype(jnp.int32).reshape(NW, BW, L), (0, 2, 1)
    )

    mesh = plsc.VectorSubcoreMesh(core_axis_name="c", subcore_axis_name="s")

    @functools.partial(
        pl.kernel,
        mesh=mesh,
        compiler_params=pltpu.CompilerParams(
            use_tc_tiling_on_sc=False, needs_layout_passes=False
        ),
        out_type=jax.ShapeDtypeStruct((L, CT, NW, 8, LANES * 8), jnp.float32),
        scratch_types=[
            pltpu.VMEM((L, BW), jnp.int32),
            pltpu.VMEM((NBUF, G * BW, DIM), jnp.float32),
            pltpu.VMEM((XBUF, G, DIM, BW), jnp.float32),
        ]
        + [pltpu.SemaphoreType.DMA] * (NBUF + XBUF),
    )
    def emb_kernel(idx_hbm, tab_hbm, out_hbm, idx_v, raw_v, xout_v, *sems):
        gsem = sems[:NBUF]
        osem = sems[NBUF:]
        wid = lax.axis_index("s") * NC + lax.axis_index("c")
        pltpu.sync_copy(idx_hbm.at[wid], idx_v)
        lane_iota = lax.iota(jnp.int32, LANES)

        def gather(c, b):
            return [
                pltpu.make_async_copy(
                    tab_hbm.at[idx_v.at[c * G + j]],
                    raw_v.at[b].at[pl.ds(j * BW, BW)],
                    gsem[b],
                )
                for j in range(G)
            ]

        def slab_stores(c, xb):
            return [
                pltpu.make_async_copy(
                    xout_v.at[xb].at[j].at[pl.ds(ct * 8, 8)],
                    out_hbm.at[c * G + j, ct, wid],
                    osem[xb],
                )
                for j in range(G)
                for ct in range(CT)
            ]

        for cp in gather(0, 0):
            cp.start()
        for cp in gather(1, 1):
            cp.start()

        @pl.loop(0, nt)
        def outer(t):
            for b in range(NBUF):
                c = t * NBUF + b
                xb = b % XBUF
                if b < 2:
                    for cp in gather(c + 2, (b + 2) % NBUF):
                        cp.start()
                else:
                    @pl.when(t < nt - 1)
                    def _():
                        for cp in gather(c + 2, (b + 2) % NBUF):
                            cp.start()

                for cp in gather(c, b):
                    cp.wait()

                if b < 2:
                    @pl.when(t > 0)
                    def _():
                        for cp in slab_stores(c - 2, xb):
                            cp.wait()
                else:
                    for cp in slab_stores(c - 2, xb):
                        cp.wait()

                @plsc.parallel_loop(0, LANES * G * 32, 1, unroll=8)
                def transpose_scale(q):
                    s = q & 15
                    blk = q >> 4
                    j = blk >> 5
                    rem = blk & 31
                    c0 = (rem >> 3) * LANES
                    r0 = (rem & 7) * LANES
                    rot = (lane_iota + s) & 15
                    rows = lane_iota + (r0 + j * BW)
                    cols = rot + c0
                    v = plsc.load_gather(raw_v.at[b], [rows, cols])
                    plsc.store_scatter(
                        xout_v.at[xb],
                        [cols * 0 + j, cols, lane_iota + r0],
                        v * SCALE,
                    )

                for cp in slab_stores(c, xb):
                    cp.start()

        for cc, xb in ((nch - 2, 0), (nch - 1, 1)):
            for cp in slab_stores(cc, xb):
                cp.wait()

    out5 = emb_kernel(idx3, table)
    return jnp.transpose(out5, (2, 4, 0, 1, 3)).reshape(B, L, DIM)
